# baseline (speedup 1.0000x reference)
import functools

import jax
import jax.numpy as jnp
from jax import lax
from jax.experimental import pallas as pl
from jax.experimental.pallas import tpu as pltpu
from jax.experimental.pallas import tpu_sc as plsc

EMBED = 16
N_COLS = 19
BATCH_C = 4096
N_PROD = 10000
N_PROD_PAD = 10240
NC, NS = 2, 16
NW = NC * NS
C_PER_W = BATCH_C // NW
P_PER_W = N_PROD_PAD // NW
FEAT = N_COLS * EMBED

def _sc_gather_body(cidx, pidx, t_cust, t_prod,
               t_fn, t_active, t_club, t_fnews, t_age, t_postal, t_price,
               t_schan, t_season, t_day, t_month, t_year, t_pname, t_ptype,
               t_graph, t_colour, t_dept, t_index,
               ce_out, pe_out, idx_c, idx_p, buf, sem):
    shared = [t_fn, t_active, t_club, t_fnews, t_age, t_postal, t_price,
              t_schan, t_season, t_day, t_month, t_year, t_pname, t_ptype,
              t_graph, t_colour, t_dept, t_index]
    cust_tables = [t_cust] + shared
    prod_tables = [t_prod] + shared

    wid = lax.axis_index("s") * NC + lax.axis_index("c")
    cbase = wid * C_PER_W
    pbase = wid * P_PER_W

    pltpu.sync_copy(cidx.at[:, pl.ds(cbase, C_PER_W)], idx_c)
    pltpu.sync_copy(pidx.at[:, pl.ds(pbase, P_PER_W)], idx_p)

    cps = [
        pltpu.async_copy(cust_tables[i].at[idx_c.at[i]],
                         buf.at[i, pl.ds(0, C_PER_W)], sem)
        for i in range(N_COLS)
    ]
    for cp in cps:
        cp.wait()
    wps = [
        pltpu.async_copy(buf.at[i, pl.ds(0, C_PER_W)],
                         ce_out.at[pl.ds(cbase, C_PER_W), pl.ds(i * EMBED, EMBED)],
                         sem)
        for i in range(N_COLS)
    ]
    for wp in wps:
        wp.wait()

    gps = [
        pltpu.async_copy(prod_tables[i].at[idx_p.at[i]], buf.at[i], sem)
        for i in range(N_COLS)
    ]
    for gp in gps:
        gp.wait()
    wps = [
        pltpu.async_copy(buf.at[i],
                         pe_out.at[pl.ds(pbase, P_PER_W), pl.ds(i * EMBED, EMBED)],
                         sem)
        for i in range(N_COLS)
    ]
    for wp in wps:
        wp.wait()


@functools.cache
def _sc_gather_kernel():
    mesh = plsc.VectorSubcoreMesh(core_axis_name="c", subcore_axis_name="s")
    return pl.kernel(
        _sc_gather_body,
        mesh=mesh,
        out_type=[
            jax.ShapeDtypeStruct((BATCH_C, FEAT), jnp.float32),
            jax.ShapeDtypeStruct((N_PROD_PAD, FEAT), jnp.float32),
        ],
        scratch_types=[
            pltpu.VMEM((N_COLS, C_PER_W), jnp.int32),
            pltpu.VMEM((N_COLS, P_PER_W), jnp.int32),
            pltpu.VMEM((N_COLS, P_PER_W, EMBED), jnp.float32),
            pltpu.SemaphoreType.DMA,
        ],
        compiler_params=pltpu.CompilerParams(use_tc_tiling_on_sc=False),
    )


def _mm_body(ce_ref, pe_ref, o_ref):
    o_ref[...] = lax.dot_general(
        ce_ref[...], pe_ref[...],
        (((1,), (1,)), ((), ())),
        preferred_element_type=jnp.float32,
    )


_BM, _BN = 1024, 2048


def _matmul(ce, pe):
    return pl.pallas_call(
        _mm_body,
        grid=(BATCH_C // _BM, (N_PROD + _BN - 1) // _BN),
        in_specs=[
            pl.BlockSpec((_BM, FEAT), lambda i, j: (i, 0)),
            pl.BlockSpec((_BN, FEAT), lambda i, j: (j, 0)),
        ],
        out_specs=pl.BlockSpec((_BM, _BN), lambda i, j: (i, j)),
        out_shape=jax.ShapeDtypeStruct((BATCH_C, N_PROD), jnp.float32),
    )(ce, pe)


def kernel(Customer_data, Product_data, W_customer, W_product, W_price, W_age,
           W_colour, W_department, W_prod_name, W_prod_type, W_index,
           W_sales_channel, W_season, W_day, W_month, W_year, W_FN, W_Active,
           W_club, W_fashion_news, W_postal, W_graphical):
    cidx = Customer_data.astype(jnp.int32).T
    pidx = jnp.pad(Product_data.astype(jnp.int32),
                   ((0, N_PROD_PAD - N_PROD), (0, 0))).T
    shared = (W_FN, W_Active, W_club, W_fashion_news, W_age, W_postal,
              W_price, W_sales_channel, W_season, W_day, W_month, W_year,
              W_prod_name, W_prod_type, W_graphical, W_colour, W_department,
              W_index)
    ce, pe = _sc_gather_kernel()(cidx, pidx, W_customer, W_product, *shared)
    return _matmul(ce, pe)


# speedup vs baseline: 1.0127x; 1.0127x over previous
import functools

import jax
import jax.numpy as jnp
from jax import lax
from jax.experimental import pallas as pl
from jax.experimental.pallas import tpu as pltpu
from jax.experimental.pallas import tpu_sc as plsc

EMBED = 16
N_COLS = 19
BATCH_C = 4096
N_PROD = 10000
N_PROD_PAD = 10240
NC, NS = 2, 16
NW = NC * NS
C_PER_W = BATCH_C // NW
P_PER_W = N_PROD_PAD // NW
FEAT = N_COLS * EMBED

def _sc_gather_body(cidx, pidx, t_cust, t_prod,
               t_fn, t_active, t_club, t_fnews, t_age, t_postal, t_price,
               t_schan, t_season, t_day, t_month, t_year, t_pname, t_ptype,
               t_graph, t_colour, t_dept, t_index,
               ce_out, pe_out, idx_c, idx_p, buf, sem):
    shared = [t_fn, t_active, t_club, t_fnews, t_age, t_postal, t_price,
              t_schan, t_season, t_day, t_month, t_year, t_pname, t_ptype,
              t_graph, t_colour, t_dept, t_index]
    cust_tables = [t_cust] + shared
    prod_tables = [t_prod] + shared

    wid = lax.axis_index("s") * NC + lax.axis_index("c")
    cbase = wid * C_PER_W
    pbase = wid * P_PER_W

    pltpu.sync_copy(cidx.at[:, pl.ds(cbase, C_PER_W)], idx_c)
    pltpu.sync_copy(pidx.at[:, pl.ds(pbase, P_PER_W)], idx_p)

    cps = [
        pltpu.async_copy(cust_tables[i].at[idx_c.at[i]],
                         buf.at[i, pl.ds(0, C_PER_W)], sem)
        for i in range(N_COLS)
    ]
    for cp in cps:
        cp.wait()
    wps = [
        pltpu.async_copy(buf.at[i, pl.ds(0, C_PER_W)],
                         ce_out.at[pl.ds(cbase, C_PER_W), pl.ds(i * EMBED, EMBED)],
                         sem)
        for i in range(N_COLS)
    ]
    for wp in wps:
        wp.wait()

    gps = [
        pltpu.async_copy(prod_tables[i].at[idx_p.at[i]], buf.at[i], sem)
        for i in range(N_COLS)
    ]
    for gp in gps:
        gp.wait()
    wps = [
        pltpu.async_copy(buf.at[i],
                         pe_out.at[pl.ds(pbase, P_PER_W), pl.ds(i * EMBED, EMBED)],
                         sem)
        for i in range(N_COLS)
    ]
    for wp in wps:
        wp.wait()


@functools.cache
def _sc_gather_kernel():
    mesh = plsc.VectorSubcoreMesh(core_axis_name="c", subcore_axis_name="s")
    return pl.kernel(
        _sc_gather_body,
        mesh=mesh,
        out_type=[
            jax.ShapeDtypeStruct((BATCH_C, FEAT), jnp.float32),
            jax.ShapeDtypeStruct((N_PROD_PAD, FEAT), jnp.float32),
        ],
        scratch_types=[
            pltpu.VMEM((N_COLS, C_PER_W), jnp.int32),
            pltpu.VMEM((N_COLS, P_PER_W), jnp.int32),
            pltpu.VMEM((N_COLS, P_PER_W, EMBED), jnp.float32),
            pltpu.SemaphoreType.DMA,
        ],
        compiler_params=pltpu.CompilerParams(use_tc_tiling_on_sc=False),
    )


def _mm_body(ce_ref, pe_ref, o_ref):
    o_ref[...] = lax.dot_general(
        ce_ref[...], pe_ref[...],
        (((1,), (1,)), ((), ())),
        preferred_element_type=jnp.float32,
    )


_BM, _BN = 1024, 2048


def _matmul(ce, pe):
    return pl.pallas_call(
        _mm_body,
        grid=(BATCH_C // _BM, (N_PROD + _BN - 1) // _BN),
        in_specs=[
            pl.BlockSpec((_BM, FEAT), lambda i, j: (i, 0)),
            pl.BlockSpec((_BN, FEAT), lambda i, j: (j, 0)),
        ],
        out_specs=pl.BlockSpec((_BM, _BN), lambda i, j: (i, j)),
        out_shape=jax.ShapeDtypeStruct((BATCH_C, N_PROD), jnp.float32),
    )(ce, pe)


def kernel(Customer_data, Product_data, W_customer, W_product, W_price, W_age,
           W_colour, W_department, W_prod_name, W_prod_type, W_index,
           W_sales_channel, W_season, W_day, W_month, W_year, W_FN, W_Active,
           W_club, W_fashion_news, W_postal, W_graphical):
    cidx = Customer_data.astype(jnp.int32).T
    pidx = jnp.pad(Product_data.astype(jnp.int32),
                   ((0, N_PROD_PAD - N_PROD), (0, 0))).T
    shared = (W_FN, W_Active, W_club, W_fashion_news, W_age, W_postal,
              W_price, W_sales_channel, W_season, W_day, W_month, W_year,
              W_prod_name, W_prod_type, W_graphical, W_colour, W_department,
              W_index)
    ce, pe = _sc_gather_kernel()(cidx, pidx, W_customer, W_product, *shared)
    return _matmul(ce.astype(jnp.bfloat16), pe.astype(jnp.bfloat16))


# speedup vs baseline: 1.0456x; 1.0325x over previous
import functools

import jax
import jax.numpy as jnp
from jax import lax
from jax.experimental import pallas as pl
from jax.experimental.pallas import tpu as pltpu
from jax.experimental.pallas import tpu_sc as plsc

EMBED = 16
N_COLS = 19
BATCH_C = 4096
N_PROD = 10000
N_PROD_PAD = 10240
NC, NS = 2, 16
NW = NC * NS
C_PER_W = BATCH_C // NW
P_PER_W = N_PROD_PAD // NW
FEAT = N_COLS * EMBED


def _gather_side_body(rows_per_w, idxT, tables, out, idx_v, buf, sem):
    wid = lax.axis_index("s") * NC + lax.axis_index("c")
    base = wid * rows_per_w
    pltpu.sync_copy(idxT.at[:, pl.ds(base, rows_per_w)], idx_v)
    gps = [
        pltpu.async_copy(tables[i].at[idx_v.at[i]], buf.at[i], sem)
        for i in range(N_COLS)
    ]
    for gp in gps:
        gp.wait()
    wps = [
        pltpu.async_copy(
            buf.at[i],
            out.at[pl.ds(base, rows_per_w), pl.ds(i * EMBED, EMBED)], sem)
        for i in range(N_COLS)
    ]
    for wp in wps:
        wp.wait()


def _ce_body(cidx, t0, *rest):
    tables = [t0] + list(rest[:N_COLS - 1])
    out, idx_v, buf, sem = rest[N_COLS - 1:]
    _gather_side_body(C_PER_W, cidx, tables, out, idx_v, buf, sem)


def _pe_body(pidx, t0, *rest):
    tables = [t0] + list(rest[:N_COLS - 1])
    out, idx_v, buf, sem = rest[N_COLS - 1:]
    _gather_side_body(P_PER_W, pidx, tables, out, idx_v, buf, sem)


@functools.cache
def _gather_kernels():
    mesh = plsc.VectorSubcoreMesh(core_axis_name="c", subcore_axis_name="s")
    ce_k = pl.kernel(
        _ce_body,
        mesh=mesh,
        out_type=[jax.ShapeDtypeStruct((BATCH_C, FEAT), jnp.float32)],
        scratch_types=[
            pltpu.VMEM((N_COLS, C_PER_W), jnp.int32),
            pltpu.VMEM((N_COLS, C_PER_W, EMBED), jnp.float32),
            pltpu.SemaphoreType.DMA,
        ],
        compiler_params=pltpu.CompilerParams(use_tc_tiling_on_sc=False),
    )
    pe_k = pl.kernel(
        _pe_body,
        mesh=mesh,
        out_type=[jax.ShapeDtypeStruct((N_PROD_PAD, FEAT), jnp.float32)],
        scratch_types=[
            pltpu.VMEM((N_COLS, P_PER_W), jnp.int32),
            pltpu.VMEM((N_COLS, P_PER_W, EMBED), jnp.float32),
            pltpu.SemaphoreType.DMA,
        ],
        compiler_params=pltpu.CompilerParams(use_tc_tiling_on_sc=False),
    )
    return ce_k, pe_k


def _mm_body(ce_ref, pet_ref, o_ref):
    o_ref[...] = lax.dot_general(
        ce_ref[...], pet_ref[...],
        (((1,), (0,)), ((), ())),
        preferred_element_type=jnp.float32,
    )


_BM, _BN = 1024, 2048


def _matmul(ce, pet):
    return pl.pallas_call(
        _mm_body,
        grid=(BATCH_C // _BM, (N_PROD + _BN - 1) // _BN),
        in_specs=[
            pl.BlockSpec((_BM, FEAT), lambda i, j: (i, 0)),
            pl.BlockSpec((FEAT, _BN), lambda i, j: (0, j)),
        ],
        out_specs=pl.BlockSpec((_BM, _BN), lambda i, j: (i, j)),
        out_shape=jax.ShapeDtypeStruct((BATCH_C, N_PROD), jnp.float32),
    )(ce, pet)


def kernel(Customer_data, Product_data, W_customer, W_product, W_price, W_age,
           W_colour, W_department, W_prod_name, W_prod_type, W_index,
           W_sales_channel, W_season, W_day, W_month, W_year, W_FN, W_Active,
           W_club, W_fashion_news, W_postal, W_graphical):
    cidx = Customer_data.astype(jnp.int32).T
    pidx = jnp.pad(Product_data.astype(jnp.int32),
                   ((0, N_PROD_PAD - N_PROD), (0, 0))).T
    shared = (W_FN, W_Active, W_club, W_fashion_news, W_age, W_postal,
              W_price, W_sales_channel, W_season, W_day, W_month, W_year,
              W_prod_name, W_prod_type, W_graphical, W_colour, W_department,
              W_index)
    ce_k, pe_k = _gather_kernels()
    (pe,) = pe_k(pidx, W_product, *shared)
    (ce,) = ce_k(cidx, W_customer, *shared)
    return _matmul(ce.astype(jnp.bfloat16), pe.T.astype(jnp.bfloat16))


# speedup vs baseline: 1.0470x; 1.0013x over previous
import functools

import jax
import jax.numpy as jnp
from jax import lax
from jax.experimental import pallas as pl
from jax.experimental.pallas import tpu as pltpu
from jax.experimental.pallas import tpu_sc as plsc

EMBED = 16
N_COLS = 19
BATCH_C = 4096
N_PROD = 10000
N_PROD_PAD = 10240
NC, NS = 2, 16
NW = NC * NS
C_PER_W = BATCH_C // NW
P_PER_W = N_PROD_PAD // NW
FEAT = N_COLS * EMBED


def _gather_side_body(rows_per_w, idxT, tables, out, idx_v, buf, sem, semw):
    wid = lax.axis_index("s") * NC + lax.axis_index("c")
    base = wid * rows_per_w
    pltpu.sync_copy(idxT.at[:, pl.ds(base, rows_per_w)], idx_v)
    gps = [
        pltpu.async_copy(tables[i].at[idx_v.at[i]], buf.at[i], sem)
        for i in range(N_COLS)
    ]
    wps = []
    for i in range(N_COLS):
        gps[i].wait()
        wps.append(pltpu.async_copy(
            buf.at[i],
            out.at[pl.ds(base, rows_per_w), pl.ds(i * EMBED, EMBED)], semw))
    for wp in wps:
        wp.wait()


def _ce_body(cidx, t0, *rest):
    tables = [t0] + list(rest[:N_COLS - 1])
    out, idx_v, buf, sem, semw = rest[N_COLS - 1:]
    _gather_side_body(C_PER_W, cidx, tables, out, idx_v, buf, sem, semw)


def _pe_body(pidx, t0, *rest):
    tables = [t0] + list(rest[:N_COLS - 1])
    out, idx_v, buf, sem, semw = rest[N_COLS - 1:]
    _gather_side_body(P_PER_W, pidx, tables, out, idx_v, buf, sem, semw)


@functools.cache
def _gather_kernels():
    mesh = plsc.VectorSubcoreMesh(core_axis_name="c", subcore_axis_name="s")
    ce_k = pl.kernel(
        _ce_body,
        mesh=mesh,
        out_type=[jax.ShapeDtypeStruct((BATCH_C, FEAT), jnp.float32)],
        scratch_types=[
            pltpu.VMEM((N_COLS, C_PER_W), jnp.int32),
            pltpu.VMEM((N_COLS, C_PER_W, EMBED), jnp.float32),
            pltpu.SemaphoreType.DMA,
            pltpu.SemaphoreType.DMA,
        ],
        compiler_params=pltpu.CompilerParams(use_tc_tiling_on_sc=False),
    )
    pe_k = pl.kernel(
        _pe_body,
        mesh=mesh,
        out_type=[jax.ShapeDtypeStruct((N_PROD_PAD, FEAT), jnp.float32)],
        scratch_types=[
            pltpu.VMEM((N_COLS, P_PER_W), jnp.int32),
            pltpu.VMEM((N_COLS, P_PER_W, EMBED), jnp.float32),
            pltpu.SemaphoreType.DMA,
            pltpu.SemaphoreType.DMA,
        ],
        compiler_params=pltpu.CompilerParams(use_tc_tiling_on_sc=False),
    )
    return ce_k, pe_k


def _mm_body(ce_ref, pet_ref, o_ref):
    o_ref[...] = lax.dot_general(
        ce_ref[...], pet_ref[...],
        (((1,), (0,)), ((), ())),
        preferred_element_type=jnp.float32,
    )


_BM, _BN = 1024, 2048


def _matmul(ce, pet):
    return pl.pallas_call(
        _mm_body,
        grid=(BATCH_C // _BM, (N_PROD + _BN - 1) // _BN),
        in_specs=[
            pl.BlockSpec((_BM, FEAT), lambda i, j: (i, 0)),
            pl.BlockSpec((FEAT, _BN), lambda i, j: (0, j)),
        ],
        out_specs=pl.BlockSpec((_BM, _BN), lambda i, j: (i, j)),
        out_shape=jax.ShapeDtypeStruct((BATCH_C, N_PROD), jnp.float32),
    )(ce, pet)


def kernel(Customer_data, Product_data, W_customer, W_product, W_price, W_age,
           W_colour, W_department, W_prod_name, W_prod_type, W_index,
           W_sales_channel, W_season, W_day, W_month, W_year, W_FN, W_Active,
           W_club, W_fashion_news, W_postal, W_graphical):
    cidx = Customer_data.astype(jnp.int32).T
    pidx = jnp.pad(Product_data.astype(jnp.int32),
                   ((0, N_PROD_PAD - N_PROD), (0, 0))).T
    shared = (W_FN, W_Active, W_club, W_fashion_news, W_age, W_postal,
              W_price, W_sales_channel, W_season, W_day, W_month, W_year,
              W_prod_name, W_prod_type, W_graphical, W_colour, W_department,
              W_index)
    ce_k, pe_k = _gather_kernels()
    (pe,) = pe_k(pidx, W_product, *shared)
    (ce,) = ce_k(cidx, W_customer, *shared)
    return _matmul(ce.astype(jnp.bfloat16), pe.T.astype(jnp.bfloat16))


# speedup vs baseline: 1.0494x; 1.0024x over previous
import functools

import jax
import jax.numpy as jnp
from jax import lax
from jax.experimental import pallas as pl
from jax.experimental.pallas import tpu as pltpu
from jax.experimental.pallas import tpu_sc as plsc

EMBED = 16
N_COLS = 19
BATCH_C = 4096
N_PROD = 10000
N_PROD_PAD = 10240
NC, NS = 2, 16
NW = NC * NS
C_PER_W = BATCH_C // NW
P_PER_W = N_PROD_PAD // NW
FEAT = N_COLS * EMBED


def _gather_side_body(rows_per_w, idxT, tables, out, idx_v, buf, sem, semw):
    wid = lax.axis_index("s") * NC + lax.axis_index("c")
    base = wid * rows_per_w
    pltpu.sync_copy(idxT.at[:, pl.ds(base, rows_per_w)], idx_v)
    order = [0, 6, 13] + [i for i in range(N_COLS) if i not in (0, 6, 13)]
    gps = [
        pltpu.async_copy(tables[i].at[idx_v.at[i]], buf.at[i], sem)
        for i in order
    ]
    for gp in gps:
        gp.wait()
    wps = [
        pltpu.async_copy(
            buf.at[i],
            out.at[pl.ds(base, rows_per_w), pl.ds(i * EMBED, EMBED)], semw)
        for i in order
    ]
    for wp in wps:
        wp.wait()


def _ce_body(cidx, t0, *rest):
    tables = [t0] + list(rest[:N_COLS - 1])
    out, idx_v, buf, sem, semw = rest[N_COLS - 1:]
    _gather_side_body(C_PER_W, cidx, tables, out, idx_v, buf, sem, semw)


def _pe_body(pidx, t0, *rest):
    tables = [t0] + list(rest[:N_COLS - 1])
    out, idx_v, buf, sem, semw = rest[N_COLS - 1:]
    _gather_side_body(P_PER_W, pidx, tables, out, idx_v, buf, sem, semw)


@functools.cache
def _gather_kernels():
    mesh = plsc.VectorSubcoreMesh(core_axis_name="c", subcore_axis_name="s")
    ce_k = pl.kernel(
        _ce_body,
        mesh=mesh,
        out_type=[jax.ShapeDtypeStruct((BATCH_C, FEAT), jnp.float32)],
        scratch_types=[
            pltpu.VMEM((N_COLS, C_PER_W), jnp.int32),
            pltpu.VMEM((N_COLS, C_PER_W, EMBED), jnp.float32),
            pltpu.SemaphoreType.DMA,
            pltpu.SemaphoreType.DMA,
        ],
        compiler_params=pltpu.CompilerParams(use_tc_tiling_on_sc=False),
    )
    pe_k = pl.kernel(
        _pe_body,
        mesh=mesh,
        out_type=[jax.ShapeDtypeStruct((N_PROD_PAD, FEAT), jnp.float32)],
        scratch_types=[
            pltpu.VMEM((N_COLS, P_PER_W), jnp.int32),
            pltpu.VMEM((N_COLS, P_PER_W, EMBED), jnp.float32),
            pltpu.SemaphoreType.DMA,
            pltpu.SemaphoreType.DMA,
        ],
        compiler_params=pltpu.CompilerParams(use_tc_tiling_on_sc=False),
    )
    return ce_k, pe_k


def _mm_body(ce_ref, pet_ref, o_ref):
    o_ref[...] = lax.dot_general(
        ce_ref[...], pet_ref[...],
        (((1,), (0,)), ((), ())),
        preferred_element_type=jnp.float32,
    )


_BM, _BN = 1024, 2048


def _matmul(ce, pet):
    return pl.pallas_call(
        _mm_body,
        grid=(BATCH_C // _BM, (N_PROD + _BN - 1) // _BN),
        in_specs=[
            pl.BlockSpec((_BM, FEAT), lambda i, j: (i, 0)),
            pl.BlockSpec((FEAT, _BN), lambda i, j: (0, j)),
        ],
        out_specs=pl.BlockSpec((_BM, _BN), lambda i, j: (i, j)),
        out_shape=jax.ShapeDtypeStruct((BATCH_C, N_PROD), jnp.float32),
    )(ce, pet)


def kernel(Customer_data, Product_data, W_customer, W_product, W_price, W_age,
           W_colour, W_department, W_prod_name, W_prod_type, W_index,
           W_sales_channel, W_season, W_day, W_month, W_year, W_FN, W_Active,
           W_club, W_fashion_news, W_postal, W_graphical):
    cidx = Customer_data.astype(jnp.int32).T
    pidx = jnp.pad(Product_data.astype(jnp.int32),
                   ((0, N_PROD_PAD - N_PROD), (0, 0))).T
    shared = (W_FN, W_Active, W_club, W_fashion_news, W_age, W_postal,
              W_price, W_sales_channel, W_season, W_day, W_month, W_year,
              W_prod_name, W_prod_type, W_graphical, W_colour, W_department,
              W_index)
    ce_k, pe_k = _gather_kernels()
    (pe,) = pe_k(pidx, W_product, *shared)
    (ce,) = ce_k(cidx, W_customer, *shared)
    return _matmul(ce.astype(jnp.bfloat16), pe.T.astype(jnp.bfloat16))


# speedup vs baseline: 1.0510x; 1.0015x over previous
import functools

import jax
import jax.numpy as jnp
from jax import lax
from jax.experimental import pallas as pl
from jax.experimental.pallas import tpu as pltpu
from jax.experimental.pallas import tpu_sc as plsc

EMBED = 16
N_COLS = 19
BATCH_C = 4096
N_PROD = 10000
N_PROD_PAD = 10240
NC, NS = 2, 16
NW = NC * NS
C_PER_W = BATCH_C // NW
P_PER_W = N_PROD_PAD // NW
FEAT = N_COLS * EMBED


def _gather_side_body(rows_per_w, idxT, tables, out, idx_v, buf, sem, semw):
    wid = lax.axis_index("s") * NC + lax.axis_index("c")
    base = wid * rows_per_w
    pltpu.sync_copy(idxT.at[:, pl.ds(base, rows_per_w)], idx_v)
    order = [0, 6, 13] + [i for i in range(N_COLS) if i not in (0, 6, 13)]
    gps = [
        pltpu.async_copy(tables[i].at[idx_v.at[i]], buf.at[i], sem)
        for i in order
    ]
    for gp in gps:
        gp.wait()
    wps = [
        pltpu.async_copy(
            buf.at[i],
            out.at[pl.ds(base, rows_per_w), pl.ds(i * EMBED, EMBED)], semw)
        for i in order
    ]
    for wp in wps:
        wp.wait()


def _ce_body(cidx, t0, *rest):
    tables = [t0] + list(rest[:N_COLS - 1])
    out, idx_v, buf, sem, semw = rest[N_COLS - 1:]
    _gather_side_body(C_PER_W, cidx, tables, out, idx_v, buf, sem, semw)


def _pe_body(pidx, t0, *rest):
    tables = [t0] + list(rest[:N_COLS - 1])
    out, idx_v, buf, sem, semw = rest[N_COLS - 1:]
    _gather_side_body(P_PER_W, pidx, tables, out, idx_v, buf, sem, semw)


@functools.cache
def _gather_kernels():
    mesh = plsc.VectorSubcoreMesh(core_axis_name="c", subcore_axis_name="s")
    ce_k = pl.kernel(
        _ce_body,
        mesh=mesh,
        out_type=[jax.ShapeDtypeStruct((BATCH_C, FEAT), jnp.float32)],
        scratch_types=[
            pltpu.VMEM((N_COLS, C_PER_W), jnp.int32),
            pltpu.VMEM((N_COLS, C_PER_W, EMBED), jnp.float32),
            pltpu.SemaphoreType.DMA,
            pltpu.SemaphoreType.DMA,
        ],
        compiler_params=pltpu.CompilerParams(use_tc_tiling_on_sc=False),
    )
    pe_k = pl.kernel(
        _pe_body,
        mesh=mesh,
        out_type=[jax.ShapeDtypeStruct((N_PROD_PAD, FEAT), jnp.float32)],
        scratch_types=[
            pltpu.VMEM((N_COLS, P_PER_W), jnp.int32),
            pltpu.VMEM((N_COLS, P_PER_W, EMBED), jnp.float32),
            pltpu.SemaphoreType.DMA,
            pltpu.SemaphoreType.DMA,
        ],
        compiler_params=pltpu.CompilerParams(use_tc_tiling_on_sc=False),
    )
    return ce_k, pe_k


def _mm_body(ce_ref, pet_ref, o_ref):
    o_ref[...] = lax.dot_general(
        ce_ref[...], pet_ref[...],
        (((1,), (0,)), ((), ())),
        preferred_element_type=jnp.float32,
    )


_BM, _BN = 2048, 2048


def _matmul(ce, pet):
    return pl.pallas_call(
        _mm_body,
        grid=(BATCH_C // _BM, (N_PROD + _BN - 1) // _BN),
        in_specs=[
            pl.BlockSpec((_BM, FEAT), lambda i, j: (i, 0)),
            pl.BlockSpec((FEAT, _BN), lambda i, j: (0, j)),
        ],
        out_specs=pl.BlockSpec((_BM, _BN), lambda i, j: (i, j)),
        out_shape=jax.ShapeDtypeStruct((BATCH_C, N_PROD), jnp.float32),
    )(ce, pet)


def kernel(Customer_data, Product_data, W_customer, W_product, W_price, W_age,
           W_colour, W_department, W_prod_name, W_prod_type, W_index,
           W_sales_channel, W_season, W_day, W_month, W_year, W_FN, W_Active,
           W_club, W_fashion_news, W_postal, W_graphical):
    cidx = Customer_data.astype(jnp.int32).T
    pidx = jnp.pad(Product_data.astype(jnp.int32),
                   ((0, N_PROD_PAD - N_PROD), (0, 0))).T
    shared = (W_FN, W_Active, W_club, W_fashion_news, W_age, W_postal,
              W_price, W_sales_channel, W_season, W_day, W_month, W_year,
              W_prod_name, W_prod_type, W_graphical, W_colour, W_department,
              W_index)
    ce_k, pe_k = _gather_kernels()
    (pe,) = pe_k(pidx, W_product, *shared)
    (ce,) = ce_k(cidx, W_customer, *shared)
    return _matmul(ce.astype(jnp.bfloat16), pe.T.astype(jnp.bfloat16))
